# Initial kernel scaffold; baseline (speedup 1.0000x reference)
import jax
import jax.numpy as jnp
from jax.experimental import pallas as pl


def kernel(x, edge_index, eps0, eps1, eps2, W0a, b0a, g0, be0, W0b, b0b, W1a, b1a, W1b, b1b, W2a, b2a, W2b, b2b, ng0, nb0, ng1, nb1, ng2, nb2):
    raise NotImplementedError("write your pallas kernel here")


# speedup vs baseline: 3.8692x; 3.8692x over previous
import functools

import jax
import jax.numpy as jnp
from jax import lax
from jax.experimental import pallas as pl
from jax.experimental.pallas import tpu as pltpu
from jax.experimental.pallas import tpu_sc as plsc

NC = 2
NS = 16
NW = NC * NS
K = 128
NBUF = 2


def _make_scatter(n_pad, d, chunks):
    zrows = n_pad // NS
    zblocks = [(j * K, K) for j in range(zrows // K)]
    if zrows % K:
        zblocks.append((zrows - zrows % K, zrows % K))
    mesh = plsc.VectorSubcoreMesh(core_axis_name="c", subcore_axis_name="s")

    @functools.partial(
        pl.kernel,
        mesh=mesh,
        out_type=jax.ShapeDtypeStruct((NC, n_pad, d), jnp.float32),
        scratch_types=[
            pltpu.VMEM((K,), jnp.int32),
            pltpu.VMEM((K,), jnp.int32),
            pltpu.VMEM((K,), jnp.int32),
            pltpu.VMEM((K,), jnp.int32),
            pltpu.VMEM((K, d), jnp.float32),
            pltpu.VMEM((K, d), jnp.float32),
            pltpu.VMEM((K, d), jnp.float32),
            pltpu.VMEM_SHARED((n_pad, d), jnp.float32),
            pltpu.SemaphoreType.DMA,
            pltpu.SemaphoreType.DMA,
        ],
    )
    def scatter(x_hbm, srcw_hbm, dstw_hbm, zeros_hbm, out_hbm,
                src_i0, src_i1, dst_i0, dst_i1, rows0, rows1, zbuf,
                acc, gsem0, gsem1):
        c = lax.axis_index("c")
        s = lax.axis_index("s")
        wid = s * NC + c
        src_i = [src_i0, src_i1]
        dst_i = [dst_i0, dst_i1]
        rows = [rows0, rows1]
        gsem = [gsem0, gsem1]

        pltpu.sync_copy(zeros_hbm, zbuf)
        row0 = s * zrows
        for off, sz in zblocks:
            pltpu.sync_copy(zbuf.at[pl.ds(0, sz)],
                            acc.at[pl.ds(row0 + off, sz)])
        plsc.subcore_barrier()

        def step(ci, carry):
            pltpu.sync_copy(srcw_hbm.at[wid, ci], src_i0)
            pltpu.sync_copy(dstw_hbm.at[wid, ci], dst_i0)
            pltpu.async_copy(x_hbm.at[src_i0], rows0, gsem0).wait()
            pltpu.sync_copy(rows0, acc.at[dst_i0], add=True)
            return carry

        lax.fori_loop(0, chunks, step, 0)
        plsc.subcore_barrier()

        for off, sz in zblocks:
            pltpu.sync_copy(acc.at[pl.ds(row0 + off, sz)],
                            rows0.at[pl.ds(0, sz)])
            pltpu.sync_copy(rows0.at[pl.ds(0, sz)],
                            out_hbm.at[c, pl.ds(row0 + off, sz)])

    return scatter


def _bn(t, g, b, n):
    m = jnp.sum(t, axis=0, keepdims=True) * (1.0 / n)
    ctr = t - m
    v = jnp.sum(ctr * ctr, axis=0, keepdims=True) * (1.0 / n)
    return g * ctr * lax.rsqrt(v + 1e-5) + b


def _mlp_body(scale_ref, h_ref, parts_ref, wa_ref, ba_ref, g1_ref, b1_ref,
              wb_ref, bb_ref, g2_ref, b2_ref, out_ref, *, nrows, has_bn1):
    sc = scale_ref[0, 0]
    h = (h_ref[...] * sc + parts_ref[0, :nrows, :] + parts_ref[1, :nrows, :])
    t = jnp.dot(h, wa_ref[...], preferred_element_type=jnp.float32) + ba_ref[...]
    if has_bn1:
        t = _bn(t, g1_ref[...], b1_ref[...], nrows)
    t = jnp.maximum(t, 0.0)
    u = jnp.dot(t, wb_ref[...], preferred_element_type=jnp.float32) + bb_ref[...]
    u = _bn(u, g2_ref[...], b2_ref[...], nrows)
    out_ref[...] = jnp.maximum(u, 0.0)


def _mlp(h, parts, eps, wa, ba, g1, b1, wb, bb, g2, b2, has_bn1):
    n, d = h.shape
    scale = jnp.reshape(1.0 + eps, (1, 1)).astype(jnp.float32)
    if g1 is None:
        g1 = jnp.ones((d,), jnp.float32)
        b1 = jnp.zeros((d,), jnp.float32)
    args = (scale, h, parts, wa.T, ba.reshape(1, d), g1.reshape(1, d),
            b1.reshape(1, d), wb.T, bb.reshape(1, d), g2.reshape(1, d),
            b2.reshape(1, d))
    vmem = pl.BlockSpec(memory_space=pltpu.VMEM)
    return pl.pallas_call(
        functools.partial(_mlp_body, nrows=n, has_bn1=has_bn1),
        out_shape=jax.ShapeDtypeStruct((n, d), jnp.float32),
        in_specs=[pl.BlockSpec(memory_space=pltpu.SMEM)] + [vmem] * 10,
        out_specs=vmem,
    )(*args)


def kernel(x, edge_index, eps0, eps1, eps2, W0a, b0a, g0, be0, W0b, b0b,
           W1a, b1a, W1b, b1b, W2a, b2a, W2b, b2b,
           ng0, nb0, ng1, nb1, ng2, nb2):
    n, d = x.shape
    e = edge_index.shape[1]
    ew = e // NW
    chunks = -(-ew // K)
    ewp = chunks * K
    n_pad = -(-(n + 1) // (NS * 8)) * (NS * 8)

    src = edge_index[0].reshape(NW, ew)
    dst = edge_index[1].reshape(NW, ew)
    pad = ewp - ew
    if pad:
        src = jnp.pad(src, ((0, 0), (0, pad)))
        dst = jnp.pad(dst, ((0, 0), (0, pad)), constant_values=n)
    src = src.reshape(NW, chunks, K)
    dst = dst.reshape(NW, chunks, K)
    zeros_kd = jnp.zeros((K, d), jnp.float32)

    scatter = _make_scatter(n_pad, d, chunks)

    h = x
    parts = scatter(h, src, dst, zeros_kd)
    h = _mlp(h, parts, eps0, W0a, b0a, g0, be0, W0b, b0b, ng0, nb0, True)
    parts = scatter(h, src, dst, zeros_kd)
    h = _mlp(h, parts, eps1, W1a, b1a, None, None, W1b, b1b, ng1, nb1, False)
    parts = scatter(h, src, dst, zeros_kd)
    h = _mlp(h, parts, eps2, W2a, b2a, None, None, W2b, b2b, ng2, nb2, False)
    return h


# speedup vs baseline: 5.4341x; 1.4045x over previous
import functools

import jax
import jax.numpy as jnp
from jax import lax
from jax.experimental import pallas as pl
from jax.experimental.pallas import tpu as pltpu
from jax.experimental.pallas import tpu_sc as plsc

NC = 2
NS = 16
NW = NC * NS
K = 128
NBUF = 2


def _make_scatter(n_pad, d, chunks):
    zrows = n_pad // NS
    zblocks = [(j * K, K) for j in range(zrows // K)]
    if zrows % K:
        zblocks.append((zrows - zrows % K, zrows % K))
    mesh = plsc.VectorSubcoreMesh(core_axis_name="c", subcore_axis_name="s")

    @functools.partial(
        pl.kernel,
        mesh=mesh,
        out_type=jax.ShapeDtypeStruct((NC, n_pad, d), jnp.float32),
        scratch_types=[
            pltpu.VMEM((2, K), jnp.int32),
            pltpu.VMEM((2, K), jnp.int32),
            pltpu.VMEM((K, d), jnp.float32),
            pltpu.VMEM((K, d), jnp.float32),
            pltpu.VMEM((K, d), jnp.float32),
            pltpu.VMEM_SHARED((n_pad, d), jnp.float32),
            pltpu.SemaphoreType.DMA,
            pltpu.SemaphoreType.DMA,
        ],
    )
    def scatter(x_hbm, idxw_hbm, zeros_hbm, out_hbm,
                idx0, idx1, rows0, rows1, zbuf, acc, gsem0, gsem1):
        c = lax.axis_index("c")
        s = lax.axis_index("s")
        wid = s * NC + c

        pltpu.sync_copy(zeros_hbm, zbuf)
        row0 = s * zrows
        for off, sz in zblocks:
            pltpu.sync_copy(zbuf.at[pl.ds(0, sz)],
                            acc.at[pl.ds(row0 + off, sz)])
        plsc.subcore_barrier()

        def fetch_idx(ci, idx):
            pltpu.sync_copy(idxw_hbm.at[wid, ci], idx)

        def gather_start(idx, rows, gsem):
            pltpu.make_async_copy(x_hbm.at[idx.at[0]], rows, gsem).start()

        def gather_wait(idx, rows, gsem):
            pltpu.make_async_copy(x_hbm.at[idx.at[0]], rows, gsem).wait()

        def scat(idx, rows):
            pltpu.sync_copy(rows, acc.at[idx.at[1]], add=True)

        fetch_idx(0, idx0)
        gather_start(idx0, rows0, gsem0)

        def pair(i, carry):
            ci0 = 2 * i
            fetch_idx(ci0 + 1, idx1)
            gather_start(idx1, rows1, gsem1)
            gather_wait(idx0, rows0, gsem0)
            scat(idx0, rows0)

            @pl.when(ci0 + 2 < chunks)
            def _():
                fetch_idx(ci0 + 2, idx0)
                gather_start(idx0, rows0, gsem0)

            gather_wait(idx1, rows1, gsem1)
            scat(idx1, rows1)
            return carry

        lax.fori_loop(0, chunks // 2, pair, 0)
        if chunks % 2:
            gather_wait(idx0, rows0, gsem0)
            scat(idx0, rows0)
        plsc.subcore_barrier()

        for off, sz in zblocks:
            pltpu.sync_copy(acc.at[pl.ds(row0 + off, sz)],
                            out_hbm.at[c, pl.ds(row0 + off, sz)])

    return scatter


def _bn(t, g, b, n):
    m = jnp.sum(t, axis=0, keepdims=True) * (1.0 / n)
    ctr = t - m
    v = jnp.sum(ctr * ctr, axis=0, keepdims=True) * (1.0 / n)
    return g * ctr * lax.rsqrt(v + 1e-5) + b


def _mlp_body(scale_ref, h_ref, parts_ref, wa_ref, ba_ref, g1_ref, b1_ref,
              wb_ref, bb_ref, g2_ref, b2_ref, out_ref, *, nrows, has_bn1):
    sc = scale_ref[0, 0]
    h = (h_ref[...] * sc + parts_ref[0, :nrows, :] + parts_ref[1, :nrows, :])
    t = jnp.dot(h, wa_ref[...], preferred_element_type=jnp.float32) + ba_ref[...]
    if has_bn1:
        t = _bn(t, g1_ref[...], b1_ref[...], nrows)
    t = jnp.maximum(t, 0.0)
    u = jnp.dot(t, wb_ref[...], preferred_element_type=jnp.float32) + bb_ref[...]
    u = _bn(u, g2_ref[...], b2_ref[...], nrows)
    out_ref[...] = jnp.maximum(u, 0.0)


def _mlp(h, parts, eps, wa, ba, g1, b1, wb, bb, g2, b2, has_bn1):
    n, d = h.shape
    scale = jnp.reshape(1.0 + eps, (1, 1)).astype(jnp.float32)
    if g1 is None:
        g1 = jnp.ones((d,), jnp.float32)
        b1 = jnp.zeros((d,), jnp.float32)
    args = (scale, h, parts, wa.T, ba.reshape(1, d), g1.reshape(1, d),
            b1.reshape(1, d), wb.T, bb.reshape(1, d), g2.reshape(1, d),
            b2.reshape(1, d))
    vmem = pl.BlockSpec(memory_space=pltpu.VMEM)
    return pl.pallas_call(
        functools.partial(_mlp_body, nrows=n, has_bn1=has_bn1),
        out_shape=jax.ShapeDtypeStruct((n, d), jnp.float32),
        in_specs=[pl.BlockSpec(memory_space=pltpu.SMEM)] + [vmem] * 10,
        out_specs=vmem,
    )(*args)


def kernel(x, edge_index, eps0, eps1, eps2, W0a, b0a, g0, be0, W0b, b0b,
           W1a, b1a, W1b, b1b, W2a, b2a, W2b, b2b,
           ng0, nb0, ng1, nb1, ng2, nb2):
    n, d = x.shape
    e = edge_index.shape[1]
    ew = e // NW
    chunks = -(-ew // K)
    ewp = chunks * K
    n_pad = -(-(n + 1) // (NS * 8)) * (NS * 8)

    src = edge_index[0].reshape(NW, ew)
    dst = edge_index[1].reshape(NW, ew)
    pad = ewp - ew
    if pad:
        src = jnp.pad(src, ((0, 0), (0, pad)))
        dst = jnp.pad(dst, ((0, 0), (0, pad)), constant_values=n)
    idx = jnp.stack([src.reshape(NW, chunks, K),
                     dst.reshape(NW, chunks, K)], axis=2)
    zeros_kd = jnp.zeros((K, d), jnp.float32)

    scatter = _make_scatter(n_pad, d, chunks)

    h = x
    parts = scatter(h, idx, zeros_kd)
    h = _mlp(h, parts, eps0, W0a, b0a, g0, be0, W0b, b0b, ng0, nb0, True)
    parts = scatter(h, idx, zeros_kd)
    h = _mlp(h, parts, eps1, W1a, b1a, None, None, W1b, b1b, ng1, nb1, False)
    parts = scatter(h, idx, zeros_kd)
    h = _mlp(h, parts, eps2, W2a, b2a, None, None, W2b, b2b, ng2, nb2, False)
    return h


# speedup vs baseline: 5.7823x; 1.0641x over previous
import functools

import jax
import jax.numpy as jnp
from jax import lax
from jax.experimental import pallas as pl
from jax.experimental.pallas import tpu as pltpu
from jax.experimental.pallas import tpu_sc as plsc

NC = 2
NS = 16
NW = NC * NS
K = 128
NBUF = 2


def _make_scatter(n_pad, d, chunks):
    zrows = n_pad // NS
    zblocks = [(j * K, K) for j in range(zrows // K)]
    if zrows % K:
        zblocks.append((zrows - zrows % K, zrows % K))
    mesh = plsc.VectorSubcoreMesh(core_axis_name="c", subcore_axis_name="s")

    @functools.partial(
        pl.kernel,
        mesh=mesh,
        out_type=jax.ShapeDtypeStruct((NC, n_pad, d), jnp.float32),
        scratch_types=[
            pltpu.VMEM((2, K), jnp.int32),
            pltpu.VMEM((2, K), jnp.int32),
            pltpu.VMEM((2, K), jnp.int32),
            pltpu.VMEM((2, K), jnp.int32),
            pltpu.VMEM((K, d), jnp.float32),
            pltpu.VMEM((K, d), jnp.float32),
            pltpu.VMEM_SHARED((n_pad, d), jnp.float32),
            pltpu.SemaphoreType.DMA,
            pltpu.SemaphoreType.DMA,
            pltpu.SemaphoreType.DMA,
            pltpu.SemaphoreType.DMA,
            pltpu.SemaphoreType.DMA,
            pltpu.SemaphoreType.DMA,
            pltpu.SemaphoreType.DMA,
            pltpu.SemaphoreType.DMA,
            pltpu.SemaphoreType.DMA,
        ],
    )
    def scatter(x_hbm, idxw_hbm, zeros_hbm, out_hbm,
                idx0, idx1, idx2, idx3, rows0, rows1, acc,
                is0, is1, is2, is3, gs0, gs1, ss0, ss1, bsem):
        c = lax.axis_index("c")
        s = lax.axis_index("s")
        wid = s * NC + c
        idx = [idx0, idx1, idx2, idx3]
        isem = [is0, is1, is2, is3]
        rows = [rows0, rows1]
        gsem = [gs0, gs1]
        ssem = [ss0, ss1]

        def fstart(ci, q):
            pltpu.make_async_copy(idxw_hbm.at[wid, ci], idx[q],
                                  isem[q]).start()

        def fwait(ci, q):
            pltpu.make_async_copy(idxw_hbm.at[wid, ci], idx[q],
                                  isem[q]).wait()

        def gstart(q, b):
            pltpu.make_async_copy(x_hbm.at[idx[q].at[0]], rows[b],
                                  gsem[b]).start()

        def gwait(q, b):
            pltpu.make_async_copy(x_hbm.at[idx[q].at[0]], rows[b],
                                  gsem[b]).wait()

        def sstart(q, b):
            pltpu.make_async_copy(rows[b], acc.at[idx[q].at[1]],
                                  ssem[b]).start(add=True)

        def swait(q, b):
            pltpu.make_async_copy(rows[b], acc.at[idx[q].at[1]],
                                  ssem[b]).wait()

        fstart(0, 0)
        if chunks > 1:
            fstart(1, 1)

        pltpu.sync_copy(zeros_hbm, rows0)
        row0 = s * zrows
        for off, sz in zblocks:
            pltpu.make_async_copy(rows0.at[pl.ds(0, sz)],
                                  acc.at[pl.ds(row0 + off, sz)],
                                  bsem).start()
        for off, sz in zblocks:
            pltpu.make_async_copy(rows0.at[pl.ds(0, sz)],
                                  acc.at[pl.ds(row0 + off, sz)],
                                  bsem).wait()
        plsc.subcore_barrier()

        def process(ci, j):
            q = j & 3
            b = j & 1

            @pl.when(ci + 2 < chunks)
            def _():
                fstart(ci + 2, (q + 2) & 3)

            @pl.when(ci > 0)
            def _():
                swait((q + 3) & 3, b ^ 1)

            @pl.when(ci + 1 < chunks)
            def _():
                fwait(ci + 1, (q + 1) & 3)
                gstart((q + 1) & 3, b ^ 1)

            gwait(q, b)
            sstart(q, b)

        fwait(0, 0)
        gstart(0, 0)

        def quad(i, carry):
            ci = 4 * i
            process(ci, 0)
            process(ci + 1, 1)
            process(ci + 2, 2)
            process(ci + 3, 3)
            return carry

        nq = chunks // 4
        lax.fori_loop(0, nq, quad, 0)
        for j in range(4 * nq, chunks):
            process(j, j)
        swait((chunks - 1) & 3, (chunks - 1) & 1)
        plsc.subcore_barrier()

        for off, sz in zblocks:
            pltpu.make_async_copy(acc.at[pl.ds(row0 + off, sz)],
                                  out_hbm.at[c, pl.ds(row0 + off, sz)],
                                  bsem).start()
        for off, sz in zblocks:
            pltpu.make_async_copy(acc.at[pl.ds(row0 + off, sz)],
                                  out_hbm.at[c, pl.ds(row0 + off, sz)],
                                  bsem).wait()

    return scatter


def _bn(t, g, b, n):
    m = jnp.sum(t, axis=0, keepdims=True) * (1.0 / n)
    ctr = t - m
    v = jnp.sum(ctr * ctr, axis=0, keepdims=True) * (1.0 / n)
    return g * ctr * lax.rsqrt(v + 1e-5) + b


def _mlp_body(scale_ref, h_ref, parts_ref, wa_ref, ba_ref, g1_ref, b1_ref,
              wb_ref, bb_ref, g2_ref, b2_ref, out_ref, *, nrows, has_bn1):
    sc = scale_ref[0, 0]
    h = (h_ref[...] * sc + parts_ref[0, :nrows, :] + parts_ref[1, :nrows, :])
    t = jnp.dot(h, wa_ref[...], preferred_element_type=jnp.float32) + ba_ref[...]
    if has_bn1:
        t = _bn(t, g1_ref[...], b1_ref[...], nrows)
    t = jnp.maximum(t, 0.0)
    u = jnp.dot(t, wb_ref[...], preferred_element_type=jnp.float32) + bb_ref[...]
    u = _bn(u, g2_ref[...], b2_ref[...], nrows)
    out_ref[...] = jnp.maximum(u, 0.0)


def _mlp(h, parts, eps, wa, ba, g1, b1, wb, bb, g2, b2, has_bn1):
    n, d = h.shape
    scale = jnp.reshape(1.0 + eps, (1, 1)).astype(jnp.float32)
    if g1 is None:
        g1 = jnp.ones((d,), jnp.float32)
        b1 = jnp.zeros((d,), jnp.float32)
    args = (scale, h, parts, wa.T, ba.reshape(1, d), g1.reshape(1, d),
            b1.reshape(1, d), wb.T, bb.reshape(1, d), g2.reshape(1, d),
            b2.reshape(1, d))
    vmem = pl.BlockSpec(memory_space=pltpu.VMEM)
    return pl.pallas_call(
        functools.partial(_mlp_body, nrows=n, has_bn1=has_bn1),
        out_shape=jax.ShapeDtypeStruct((n, d), jnp.float32),
        in_specs=[pl.BlockSpec(memory_space=pltpu.SMEM)] + [vmem] * 10,
        out_specs=vmem,
    )(*args)


def kernel(x, edge_index, eps0, eps1, eps2, W0a, b0a, g0, be0, W0b, b0b,
           W1a, b1a, W1b, b1b, W2a, b2a, W2b, b2b,
           ng0, nb0, ng1, nb1, ng2, nb2):
    n, d = x.shape
    e = edge_index.shape[1]
    ew = e // NW
    chunks = -(-ew // K)
    ewp = chunks * K
    n_pad = -(-(n + 1) // (NS * 8)) * (NS * 8)

    src = edge_index[0].reshape(NW, ew)
    dst = edge_index[1].reshape(NW, ew)
    pad = ewp - ew
    if pad:
        src = jnp.pad(src, ((0, 0), (0, pad)))
        dst = jnp.pad(dst, ((0, 0), (0, pad)), constant_values=n)
    idx = jnp.stack([src.reshape(NW, chunks, K),
                     dst.reshape(NW, chunks, K)], axis=2)
    zeros_kd = jnp.zeros((K, d), jnp.float32)

    scatter = _make_scatter(n_pad, d, chunks)

    h = x
    parts = scatter(h, idx, zeros_kd)
    h = _mlp(h, parts, eps0, W0a, b0a, g0, be0, W0b, b0b, ng0, nb0, True)
    parts = scatter(h, idx, zeros_kd)
    h = _mlp(h, parts, eps1, W1a, b1a, None, None, W1b, b1b, ng1, nb1, False)
    parts = scatter(h, idx, zeros_kd)
    h = _mlp(h, parts, eps2, W2a, b2a, None, None, W2b, b2b, ng2, nb2, False)
    return h
